# baseline (speedup 1.0000x reference)
import functools

import jax
import jax.numpy as jnp
from jax import lax
from jax.experimental import pallas as pl
from jax.experimental.pallas import tpu as pltpu
from jax.experimental.pallas import tpu_sc as plsc

_NW = 32


def _make_gather(B0, T, K, D, unroll=64):
    FS = 8
    NSLAB = D // FS
    NQ = _NW // NSLAB
    TOK = T // NQ
    IMG_BLK = 8
    NBLK = B0 // IMG_BLK
    NGRP = IMG_BLK * (TOK // 16)
    mesh = plsc.VectorSubcoreMesh(core_axis_name="c", subcore_axis_name="s")

    @functools.partial(
        pl.kernel,
        mesh=mesh,
        compiler_params=pltpu.CompilerParams(needs_layout_passes=False),
        out_type=jax.ShapeDtypeStruct((B0, D, T), jnp.float32),
        scratch_types=[
            pltpu.VMEM((FS, K), jnp.float32),
            pltpu.VMEM((IMG_BLK, TOK), jnp.int32),
            pltpu.VMEM((IMG_BLK, TOK), jnp.int32),
            pltpu.VMEM((IMG_BLK, FS, TOK), jnp.float32),
            pltpu.VMEM((IMG_BLK, FS, TOK), jnp.float32),
            [pltpu.SemaphoreType.DMA] * 2,
            [pltpu.SemaphoreType.DMA] * 2,
        ],
    )
    def gather_kernel(
        idx_hbm, ct_hbm, out_hbm, trows, idx0, idx1, ob0, ob1, isems, osems
    ):
        wid = lax.axis_index("s") * 2 + lax.axis_index("c")
        slab = wid // NQ
        f0 = slab * FS
        t0 = (wid % NQ) * TOK
        pltpu.sync_copy(ct_hbm.at[pl.ds(f0, FS)], trows)

        idx_bufs = (idx0, idx1)
        obufs = (ob0, ob1)

        def idx_src(img0):
            return idx_hbm.at[pl.ds(img0, IMG_BLK), pl.ds(t0, TOK)]

        def out_dst(img0):
            return out_hbm.at[
                pl.ds(img0, IMG_BLK), pl.ds(f0, FS), pl.ds(t0, TOK)
            ]

        def compute(idx_v, obuf):
            @plsc.parallel_loop(0, NGRP, unroll=unroll)
            def _grp(i, _idx_v=idx_v, _obuf=obuf):
                il = i >> 4
                g = i & 15
                iv = _idx_v[il, pl.ds(g * 16, 16)]
                zero = jnp.zeros_like(iv)
                for fl in range(FS):
                    _obuf[il, fl, pl.ds(g * 16, 16)] = plsc.load_gather(
                        trows, [zero + fl, iv]
                    )

        def step(blk, b, wait_out):
            nxt = jnp.minimum((blk + 1) * IMG_BLK, B0 - IMG_BLK)
            pltpu.async_copy(idx_src(nxt), idx_bufs[1 - b], isems[1 - b])
            pltpu.make_async_copy(idx_src(0), idx_bufs[b], isems[b]).wait()
            if wait_out:
                pltpu.make_async_copy(obufs[b], out_dst(0), osems[b]).wait()
            compute(idx_bufs[b], obufs[b])
            pltpu.async_copy(obufs[b], out_dst(blk * IMG_BLK), osems[b])

        pltpu.async_copy(idx_src(0), idx_bufs[0], isems[0])
        step(0, 0, False)
        step(1, 1, False)

        @pl.loop(1, NBLK // 2)
        def _pair(p):
            blk = p * 2
            step(blk, 0, True)
            step(blk + 1, 1, True)

        pltpu.make_async_copy(idx_src(0), idx_bufs[0], isems[0]).wait()
        pltpu.make_async_copy(obufs[0], out_dst(0), osems[0]).wait()
        pltpu.make_async_copy(obufs[1], out_dst(0), osems[1]).wait()

    return gather_kernel


def kernel(indices, codebook):
    B0, T = indices.shape
    K, D = codebook.shape
    ct = codebook.T
    out = _make_gather(B0, T, K, D)(indices, ct)
    return out.transpose(0, 2, 1)


# speedup vs baseline: 1.0357x; 1.0357x over previous
import functools

import jax
import jax.numpy as jnp
from jax import lax
from jax.experimental import pallas as pl
from jax.experimental.pallas import tpu as pltpu
from jax.experimental.pallas import tpu_sc as plsc

_NW = 32


def _make_gather(B0, T, K, D, unroll=32):
    FS = 8
    NSLAB = D // FS
    NQ = _NW // NSLAB
    TOK = T // NQ
    IMG_BLK = 8
    NBLK = B0 // IMG_BLK
    NGRP = IMG_BLK * (TOK // 16)
    mesh = plsc.VectorSubcoreMesh(core_axis_name="c", subcore_axis_name="s")

    @functools.partial(
        pl.kernel,
        mesh=mesh,
        compiler_params=pltpu.CompilerParams(needs_layout_passes=False),
        out_type=jax.ShapeDtypeStruct((B0, D, T), jnp.float32),
        scratch_types=[
            pltpu.VMEM((FS, K), jnp.float32),
            pltpu.VMEM((IMG_BLK, TOK), jnp.int32),
            pltpu.VMEM((IMG_BLK, TOK), jnp.int32),
            pltpu.VMEM((IMG_BLK, FS, TOK), jnp.float32),
            pltpu.VMEM((IMG_BLK, FS, TOK), jnp.float32),
            [pltpu.SemaphoreType.DMA] * 2,
            [pltpu.SemaphoreType.DMA] * 2,
        ],
    )
    def gather_kernel(
        idx_hbm, ct_hbm, out_hbm, trows, idx0, idx1, ob0, ob1, isems, osems
    ):
        wid = lax.axis_index("s") * 2 + lax.axis_index("c")
        slab = wid // NQ
        f0 = slab * FS
        t0 = (wid % NQ) * TOK
        pltpu.sync_copy(ct_hbm.at[pl.ds(f0, FS)], trows)

        idx_bufs = (idx0, idx1)
        obufs = (ob0, ob1)

        def idx_src(img0):
            return idx_hbm.at[pl.ds(img0, IMG_BLK), pl.ds(t0, TOK)]

        def out_dst(img0):
            return out_hbm.at[
                pl.ds(img0, IMG_BLK), pl.ds(f0, FS), pl.ds(t0, TOK)
            ]

        def compute(idx_v, obuf):
            @plsc.parallel_loop(0, NGRP, unroll=unroll)
            def _grp(i, _idx_v=idx_v, _obuf=obuf):
                il = i >> 4
                g = i & 15
                iv = _idx_v[il, pl.ds(g * 16, 16)]
                zero = jnp.zeros_like(iv)
                for fl in range(FS):
                    _obuf[il, fl, pl.ds(g * 16, 16)] = plsc.load_gather(
                        trows, [zero + fl, iv]
                    )

        def step(blk, b, wait_out):
            nxt = jnp.minimum((blk + 1) * IMG_BLK, B0 - IMG_BLK)
            pltpu.async_copy(idx_src(nxt), idx_bufs[1 - b], isems[1 - b])
            pltpu.make_async_copy(idx_src(0), idx_bufs[b], isems[b]).wait()
            if wait_out:
                pltpu.make_async_copy(obufs[b], out_dst(0), osems[b]).wait()
            compute(idx_bufs[b], obufs[b])
            pltpu.async_copy(obufs[b], out_dst(blk * IMG_BLK), osems[b])

        pltpu.async_copy(idx_src(0), idx_bufs[0], isems[0])
        step(0, 0, False)
        step(1, 1, False)

        @pl.loop(1, NBLK // 2)
        def _pair(p):
            blk = p * 2
            step(blk, 0, True)
            step(blk + 1, 1, True)

        pltpu.make_async_copy(idx_src(0), idx_bufs[0], isems[0]).wait()
        pltpu.make_async_copy(obufs[0], out_dst(0), osems[0]).wait()
        pltpu.make_async_copy(obufs[1], out_dst(0), osems[1]).wait()

    return gather_kernel


def kernel(indices, codebook):
    B0, T = indices.shape
    K, D = codebook.shape
    ct = codebook.T
    out = _make_gather(B0, T, K, D)(indices, ct)
    return out.transpose(0, 2, 1)


# speedup vs baseline: 1.0435x; 1.0075x over previous
import functools

import jax
import jax.numpy as jnp
from jax import lax
from jax.experimental import pallas as pl
from jax.experimental.pallas import tpu as pltpu
from jax.experimental.pallas import tpu_sc as plsc

_NW = 32


def _make_gather(B0, T, K, D, unroll=32):
    FS = 8
    NSLAB = D // FS
    NQ = _NW // NSLAB
    TOK = T // NQ
    IMG_BLK = 8
    NBLK = B0 // IMG_BLK
    NGRP = IMG_BLK * (TOK // 16)
    mesh = plsc.VectorSubcoreMesh(core_axis_name="c", subcore_axis_name="s")

    @functools.partial(
        pl.kernel,
        mesh=mesh,
        compiler_params=pltpu.CompilerParams(needs_layout_passes=False),
        out_type=jax.ShapeDtypeStruct((B0, D, T), jnp.float32),
        scratch_types=[
            pltpu.VMEM((FS, K), jnp.float32),
            pltpu.VMEM((IMG_BLK, TOK), jnp.int32),
            pltpu.VMEM((IMG_BLK, TOK), jnp.int32),
            pltpu.VMEM((IMG_BLK, FS, TOK), jnp.float32),
            pltpu.VMEM((IMG_BLK, FS, TOK), jnp.float32),
            [pltpu.SemaphoreType.DMA] * 2,
            [pltpu.SemaphoreType.DMA] * 2,
            pltpu.SemaphoreType.DMA,
        ],
    )
    def gather_kernel(
        idx_hbm, ct_hbm, out_hbm, trows, idx0, idx1, ob0, ob1, isems, osems, tsem
    ):
        wid = lax.axis_index("s") * 2 + lax.axis_index("c")
        slab = wid // NQ
        f0 = slab * FS
        t0 = (wid % NQ) * TOK
        tdesc = pltpu.async_copy(ct_hbm.at[pl.ds(f0, FS)], trows, tsem)

        idx_bufs = (idx0, idx1)
        obufs = (ob0, ob1)

        def idx_src(img0):
            return idx_hbm.at[pl.ds(img0, IMG_BLK), pl.ds(t0, TOK)]

        def out_dst(img0):
            return out_hbm.at[
                pl.ds(img0, IMG_BLK), pl.ds(f0, FS), pl.ds(t0, TOK)
            ]

        def compute(idx_v, obuf):
            @plsc.parallel_loop(0, NGRP, unroll=unroll)
            def _grp(i, _idx_v=idx_v, _obuf=obuf):
                il = i >> 4
                g = i & 15
                iv = _idx_v[il, pl.ds(g * 16, 16)]
                zero = jnp.zeros_like(iv)
                for fl in range(FS):
                    _obuf[il, fl, pl.ds(g * 16, 16)] = plsc.load_gather(
                        trows, [zero + fl, iv]
                    )

        def step(blk, b, wait_out):
            nxt = jnp.minimum((blk + 1) * IMG_BLK, B0 - IMG_BLK)
            pltpu.async_copy(idx_src(nxt), idx_bufs[1 - b], isems[1 - b])
            pltpu.make_async_copy(idx_src(0), idx_bufs[b], isems[b]).wait()
            if wait_out:
                pltpu.make_async_copy(obufs[b], out_dst(0), osems[b]).wait()
            compute(idx_bufs[b], obufs[b])
            pltpu.async_copy(obufs[b], out_dst(blk * IMG_BLK), osems[b])

        pltpu.async_copy(idx_src(0), idx_bufs[0], isems[0])
        tdesc.wait()
        step(0, 0, False)
        step(1, 1, False)

        @pl.loop(1, NBLK // 2)
        def _pair(p):
            blk = p * 2
            step(blk, 0, True)
            step(blk + 1, 1, True)

        pltpu.make_async_copy(idx_src(0), idx_bufs[0], isems[0]).wait()
        pltpu.make_async_copy(obufs[0], out_dst(0), osems[0]).wait()
        pltpu.make_async_copy(obufs[1], out_dst(0), osems[1]).wait()

    return gather_kernel


def kernel(indices, codebook):
    B0, T = indices.shape
    K, D = codebook.shape
    ct = codebook.T
    out = _make_gather(B0, T, K, D)(indices, ct)
    return out.transpose(0, 2, 1)
